# baseline (speedup 1.0000x reference)
import numpy as np
import jax
import jax.numpy as jnp
from jax import lax
from jax.experimental import pallas as pl
from jax.experimental.pallas import tpu as pltpu
from jax.experimental.pallas import tpu_sc as plsc

_GRID = 16
_KDIM = 384
_EPS1000 = np.float32(1000.0 * np.finfo(np.float32).eps)
_SEG = 2048
_CH = 1024
_V = 16


def _sc_prep(xs_hbm, ys_hbm, sfpx_hbm, facx_hbm, sfpy_hbm, facy_hbm, rks_hbm,
             col_v, sfp_v, fac_v, rks_v):
    wid = lax.axis_index("s") * 2 + lax.axis_index("c")
    b = wid // 2
    is_x = (wid % 2) == 0
    base = b * _SEG

    @pl.when(is_x)
    def _():
        pltpu.sync_copy(xs_hbm.at[pl.ds(base, _SEG)], col_v)

    @pl.when(jnp.logical_not(is_x))
    def _():
        pltpu.sync_copy(ys_hbm.at[pl.ds(base, _SEG)], col_v)

    def mx_body(i, acc):
        return jnp.maximum(acc, col_v[pl.ds(i * _V, _V)])
    mx = lax.fori_loop(0, _SEG // _V, mx_body, jnp.full((_V,), -1, jnp.int32))

    hv = jnp.broadcast_to(lax.reduce_max(mx, (0,)), (_V,)).astype(jnp.float32) + 1.0
    inv = jnp.full((_V,), jnp.float32(_GRID)) / hv
    ks = jnp.maximum(inv, 1.0)
    rks = jnp.full((_V,), jnp.float32(1.0)) / ks

    def tok_body(i, carry):
        c = col_v[pl.ds(i * _V, _V)].astype(jnp.float32)
        sf = (c + 0.5) * inv - 0.5
        sfp = sf * rks
        tot = jnp.zeros((_V,), jnp.float32)
        for g in range(_GRID):
            tot = tot + jnp.maximum(0.0, 1.0 - jnp.abs(sfp - jnp.float32(g) * rks))
        safe = jnp.where(tot != 0.0, tot, 1.0)
        fac = jnp.where(jnp.abs(tot) > _EPS1000, jnp.full((_V,), jnp.float32(1.0)) / safe, 0.0)
        inb = jnp.logical_and(sf >= -0.5, sf <= jnp.float32(_GRID) - 0.5)
        fac = jnp.where(inb, fac, 0.0)
        sfp_v[pl.ds(i * _V, _V)] = sfp
        fac_v[pl.ds(i * _V, _V)] = fac
        return carry
    lax.fori_loop(0, _SEG // _V, tok_body, 0)

    rks_v[...] = rks

    @pl.when(is_x)
    def _():
        pltpu.sync_copy(sfp_v, sfpx_hbm.at[pl.ds(base, _SEG)])
        pltpu.sync_copy(fac_v, facx_hbm.at[pl.ds(base, _SEG)])

    @pl.when(jnp.logical_not(is_x))
    def _():
        pltpu.sync_copy(sfp_v, sfpy_hbm.at[pl.ds(base, _SEG)])
        pltpu.sync_copy(fac_v, facy_hbm.at[pl.ds(base, _SEG)])

    pltpu.sync_copy(rks_v, rks_hbm.at[pl.ds(wid * _V, _V)])


def _sc_preprocess(xs, ys):
    n = xs.shape[0]
    mesh = plsc.VectorSubcoreMesh(core_axis_name="c", subcore_axis_name="s")
    out_type = (
        jax.ShapeDtypeStruct((n,), jnp.float32),
        jax.ShapeDtypeStruct((n,), jnp.float32),
        jax.ShapeDtypeStruct((n,), jnp.float32),
        jax.ShapeDtypeStruct((n,), jnp.float32),
        jax.ShapeDtypeStruct((32 * _V,), jnp.float32),
    )
    return pl.kernel(
        _sc_prep, out_type=out_type, mesh=mesh,
        compiler_params=pltpu.CompilerParams(needs_layout_passes=False),
        scratch_types=[
            pltpu.VMEM((_SEG,), jnp.int32),
            pltpu.VMEM((_SEG,), jnp.float32),
            pltpu.VMEM((_SEG,), jnp.float32),
            pltpu.VMEM((_V,), jnp.float32),
        ],
    )(xs, ys)


def _embed_kernel(sxp_ref, fxc_ref, syp_ref, fyc_ref, ts_ref, zs_ref,
                  rksx_ref, rksy_ref, feats_hbm, table_ref,
                  out_hbm, fbuf, obuf, insem, outsem):
    seg = sxp_ref.shape[0]
    nch = seg // _CH
    b = pl.program_id(0)

    lane_sp = jax.lax.broadcasted_iota(jnp.int32, (1, _GRID * _GRID), 1)
    lane_oh = jax.lax.broadcasted_iota(jnp.int32, (1, _KDIM - _GRID * _GRID), 1)

    rks_x = jnp.max(rksx_ref[...])
    rks_y = jnp.max(rksy_ref[...])
    il = (lane_sp // _GRID).astype(jnp.float32) * rks_x
    jl = (lane_sp % _GRID).astype(jnp.float32) * rks_y

    def copy_in(g, slot):
        return pltpu.make_async_copy(
            feats_hbm.at[pl.ds(g * _CH, _CH), :], fbuf.at[slot],
            insem.at[slot])

    def copy_out(g, slot):
        return pltpu.make_async_copy(
            obuf.at[slot], out_hbm.at[pl.ds(g * _CH, _CH), :],
            outsem.at[slot])

    nseg_ch = pl.num_programs(0) * nch

    @pl.when(b == 0)
    def _():
        copy_in(0, 0).start()

    def body(c, _):
        g = b * nch + c
        slot = jax.lax.rem(c, 2)
        nslot = 1 - slot

        @pl.when(g + 1 < nseg_ch)
        def _():
            copy_in(g + 1, nslot).start()

        r0 = c * _CH
        sxp = sxp_ref[pl.ds(r0, _CH), :]
        syp = syp_ref[pl.ds(r0, _CH), :]
        fx = fxc_ref[pl.ds(r0, _CH), :]
        fy = fyc_ref[pl.ds(r0, _CH), :]
        ts = ts_ref[pl.ds(r0, _CH), :]
        zs = zs_ref[pl.ds(r0, _CH), :]
        wxb = jnp.maximum(0.0, 1.0 - jnp.abs(sxp - il))
        wyb = jnp.maximum(0.0, 1.0 - jnp.abs(syp - jl))
        w_sp = (wxb * wyb) * (fx * fy)
        onehot = jnp.logical_or(lane_oh == ts, lane_oh - 32 == zs)
        w = jnp.concatenate([w_sp, onehot.astype(jnp.float32)], axis=1)
        acc = jax.lax.dot_general(
            w, table_ref[...], (((1,), (0,)), ((), ())),
            preferred_element_type=jnp.float32)

        copy_in(g, slot).wait()

        @pl.when(g >= 2)
        def _():
            copy_out(g - 2, slot).wait()

        obuf[slot] = fbuf[slot] + acc
        copy_out(g, slot).start()
        return 0

    jax.lax.fori_loop(0, nch, body, 0)

    @pl.when(b == pl.num_programs(0) - 1)
    def _():
        last = nseg_ch - 1
        copy_out(last - 1, jax.lax.rem(last - 1, 2)).wait()
        copy_out(last, jax.lax.rem(last, 2)).wait()


def kernel(feats, coords, cu_seqlens, pos2d_w, pos_t_w, pos_z_w):
    tot, hid = feats.shape
    nb = cu_seqlens.shape[0] - 1
    seg = tot // nb
    pad = _KDIM - (pos2d_w.shape[0] + pos_t_w.shape[0] + pos_z_w.shape[0])
    table = jnp.concatenate(
        [pos2d_w, pos_t_w, pos_z_w, jnp.zeros((pad, hid), jnp.float32)], axis=0)
    ts = coords[:, 1:2]
    zs = coords[:, 4:5]

    sfpx, facx, sfpy, facy, rks = _sc_preprocess(coords[:, 2], coords[:, 3])
    sfpx = sfpx.reshape(tot, 1)
    facx = facx.reshape(tot, 1)
    sfpy = sfpy.reshape(tot, 1)
    facy = facy.reshape(tot, 1)
    rks3 = rks.reshape(nb, 2, _V)
    rksx3 = rks3[:, 0:1, :]
    rksy3 = rks3[:, 1:2, :]

    col = pl.BlockSpec((seg, 1), lambda b: (b, 0))
    prow = pl.BlockSpec((1, 1, _V), lambda b: (b, 0, 0))
    return pl.pallas_call(
        _embed_kernel,
        grid=(nb,),
        in_specs=[
            col, col, col, col, col, col,
            prow, prow,
            pl.BlockSpec(memory_space=pltpu.MemorySpace.HBM),
            pl.BlockSpec((_KDIM, hid), lambda b: (0, 0)),
        ],
        out_specs=pl.BlockSpec(memory_space=pltpu.MemorySpace.HBM),
        out_shape=jax.ShapeDtypeStruct((tot, hid), jnp.float32),
        scratch_shapes=[
            pltpu.VMEM((2, _CH, hid), jnp.float32),
            pltpu.VMEM((2, _CH, hid), jnp.float32),
            pltpu.SemaphoreType.DMA((2,)),
            pltpu.SemaphoreType.DMA((2,)),
        ],
        compiler_params=pltpu.CompilerParams(
            dimension_semantics=("arbitrary",),
            vmem_limit_bytes=128 * 1024 * 1024,
        ),
    )(sfpx, facx, sfpy, facy, ts, zs, rksx3, rksy3, feats, table)


# speedup vs baseline: 1.3148x; 1.3148x over previous
import numpy as np
import jax
import jax.numpy as jnp
from jax.experimental import pallas as pl
from jax.experimental.pallas import tpu as pltpu

_GRID = 16
_KDIM = 384
_EPS1000 = np.float32(1000.0 * np.finfo(np.float32).eps)
_CH = 1024


def _embed_kernel(xs_ref, ys_ref, ts_ref, zs_ref, feats_hbm, table_ref,
                  out_hbm, fbuf, obuf, insem, outsem):
    seg = xs_ref.shape[0]
    nch = seg // _CH
    b = pl.program_id(0)
    row0 = b * seg

    lane_sp = jax.lax.broadcasted_iota(jnp.int32, (1, _GRID * _GRID), 1)
    lane_oh = jax.lax.broadcasted_iota(jnp.int32, (1, _KDIM - _GRID * _GRID), 1)
    lane16 = jax.lax.broadcasted_iota(jnp.int32, (1, _GRID), 1).astype(jnp.float32)

    def axis_consts(coord):
        out_size = jnp.max(coord, keepdims=True).astype(jnp.float32) + 1.0
        inv_scale = jnp.float32(_GRID) / out_size
        kscale = jnp.maximum(inv_scale, 1.0)
        rks = 1.0 / kscale
        return inv_scale, rks

    inv_x, rks_x = axis_consts(xs_ref[...])
    inv_y, rks_y = axis_consts(ys_ref[...])
    il = (lane_sp // _GRID).astype(jnp.float32) * rks_x
    jl = (lane_sp % _GRID).astype(jnp.float32) * rks_y
    l16x = lane16 * rks_x
    l16y = lane16 * rks_y

    def tok_scalars(coord, inv_scale, rks, l16):
        sf = (coord.astype(jnp.float32) + 0.5) * inv_scale - 0.5
        sfp = sf * rks
        w16 = jnp.maximum(0.0, 1.0 - jnp.abs(sfp - l16))
        tot = jnp.sum(w16, axis=1, keepdims=True)
        safe = jnp.where(tot != 0.0, tot, 1.0)
        fac = jnp.where(jnp.abs(tot) > _EPS1000, 1.0 / safe, 0.0)
        inb = jnp.logical_and(sf >= -0.5, sf <= jnp.float32(_GRID) - 0.5)
        return sfp, jnp.where(inb, fac, 0.0)

    def copy_in(g, slot):
        return pltpu.make_async_copy(
            feats_hbm.at[pl.ds(g * _CH, _CH), :], fbuf.at[slot],
            insem.at[slot])

    def copy_out(g, slot):
        return pltpu.make_async_copy(
            obuf.at[slot], out_hbm.at[pl.ds(g * _CH, _CH), :],
            outsem.at[slot])

    nseg_ch = pl.num_programs(0) * nch

    @pl.when(b == 0)
    def _():
        copy_in(0, 0).start()

    def body(c, _):
        g = b * nch + c
        slot = jax.lax.rem(c, 2)
        nslot = 1 - slot

        @pl.when(g + 1 < nseg_ch)
        def _():
            copy_in(g + 1, nslot).start()

        r0 = c * _CH
        xs = xs_ref[pl.ds(r0, _CH), :]
        ys = ys_ref[pl.ds(r0, _CH), :]
        ts = ts_ref[pl.ds(r0, _CH), :]
        zs = zs_ref[pl.ds(r0, _CH), :]
        sxp, fx = tok_scalars(xs, inv_x, rks_x, l16x)
        syp, fy = tok_scalars(ys, inv_y, rks_y, l16y)
        dx = (sxp - il).astype(jnp.bfloat16)
        dy = (syp - jl).astype(jnp.bfloat16)
        one = jnp.bfloat16(1.0)
        zero = jnp.bfloat16(0.0)
        wxb = jnp.maximum(zero, one - jnp.abs(dx))
        wyb = jnp.maximum(zero, one - jnp.abs(dy))
        w_sp = (wxb * wyb) * (fx * fy).astype(jnp.bfloat16)
        onehot = jnp.logical_or(lane_oh == ts, lane_oh - 32 == zs)
        w = jnp.concatenate([w_sp, onehot.astype(jnp.bfloat16)], axis=1)
        acc = jax.lax.dot_general(
            w, table_ref[...], (((1,), (0,)), ((), ())),
            preferred_element_type=jnp.float32)

        copy_in(g, slot).wait()

        @pl.when(g >= 2)
        def _():
            copy_out(g - 2, slot).wait()

        obuf[slot] = fbuf[slot] + acc
        copy_out(g, slot).start()
        return 0

    jax.lax.fori_loop(0, nch, body, 0)

    @pl.when(b == pl.num_programs(0) - 1)
    def _():
        last = nseg_ch - 1
        copy_out(last - 1, jax.lax.rem(last - 1, 2)).wait()
        copy_out(last, jax.lax.rem(last, 2)).wait()


def kernel(feats, coords, cu_seqlens, pos2d_w, pos_t_w, pos_z_w):
    tot, hid = feats.shape
    nb = cu_seqlens.shape[0] - 1
    seg = tot // nb
    pad = _KDIM - (pos2d_w.shape[0] + pos_t_w.shape[0] + pos_z_w.shape[0])
    table = jnp.concatenate(
        [pos2d_w, pos_t_w, pos_z_w,
         jnp.zeros((pad, hid), jnp.float32)], axis=0).astype(jnp.bfloat16)
    ts = coords[:, 1:2]
    xs = coords[:, 2:3]
    ys = coords[:, 3:4]
    zs = coords[:, 4:5]

    col = pl.BlockSpec((seg, 1), lambda b: (b, 0))
    return pl.pallas_call(
        _embed_kernel,
        grid=(nb,),
        in_specs=[
            col, col, col, col,
            pl.BlockSpec(memory_space=pltpu.MemorySpace.HBM),
            pl.BlockSpec((_KDIM, hid), lambda b: (0, 0)),
        ],
        out_specs=pl.BlockSpec(memory_space=pltpu.MemorySpace.HBM),
        out_shape=jax.ShapeDtypeStruct((tot, hid), jnp.float32),
        scratch_shapes=[
            pltpu.VMEM((2, _CH, hid), jnp.float32),
            pltpu.VMEM((2, _CH, hid), jnp.float32),
            pltpu.SemaphoreType.DMA((2,)),
            pltpu.SemaphoreType.DMA((2,)),
        ],
        compiler_params=pltpu.CompilerParams(
            dimension_semantics=("arbitrary",),
            vmem_limit_bytes=128 * 1024 * 1024,
        ),
    )(xs, ys, ts, zs, feats, table)
